# baseline (speedup 1.0000x reference)
import functools

import jax
import jax.numpy as jnp
from jax import lax
from jax.experimental import pallas as pl
from jax.experimental.pallas import tpu as pltpu
from jax.experimental.pallas import tpu_sc as plsc

N = 10000
E = 320000
IN_DIM = 128
HID = 128
OUT = 64

NPAD = 10240
K = 128
NC = 2
NS = 16
ROWS_W = 80
NB = NC * NS * ROWS_W
EPAD = NB * K
SLICE = NPAD // NS
DEG_W = 16
IB = 40
NCHUNK = ROWS_W // IB
NBUF = 2
R = 1000


def _mesh():
    return plsc.VectorSubcoreMesh(core_axis_name="c", subcore_axis_name="s")


def _sc_deg(edges3d):
    @functools.partial(
        pl.kernel,
        out_type=jax.ShapeDtypeStruct((NC, NPAD, DEG_W), jnp.float32),
        mesh=_mesh(),
        scratch_types=[
            pltpu.VMEM((ROWS_W, 2, K), jnp.int32),
            pltpu.VMEM((K, DEG_W), jnp.float32),
            pltpu.VMEM_SHARED((NPAD, DEG_W), jnp.float32),
        ],
    )
    def body(e_hbm, out_hbm, idx_v, buf_v, acc):
        c = lax.axis_index("c")
        s = lax.axis_index("s")

        @pl.loop(0, K)
        def _(i):
            buf_v[i, :] = jnp.zeros((DEG_W,), jnp.float32)

        @pl.loop(0, SLICE, step=K)
        def _(r):
            pltpu.sync_copy(buf_v, acc.at[pl.ds(s * SLICE + r, K)])

        row0 = (c * NS + s) * ROWS_W
        pltpu.sync_copy(e_hbm.at[pl.ds(row0, ROWS_W)], idx_v)

        @pl.loop(0, K)
        def _(i):
            buf_v[i, :] = jnp.ones((DEG_W,), jnp.float32)

        plsc.subcore_barrier()

        @pl.loop(0, ROWS_W)
        def _(b):
            pltpu.sync_copy(buf_v, acc.at[idx_v.at[b, 1]], add=True)

        plsc.subcore_barrier()
        pltpu.sync_copy(
            acc.at[pl.ds(s * SLICE, SLICE)],
            out_hbm.at[c].at[pl.ds(s * SLICE, SLICE)],
        )

    return body(edges3d)


def _sc_gs(hp, edges3d, d):
    @functools.partial(
        pl.kernel,
        out_type=jax.ShapeDtypeStruct((NC, NPAD, d), jnp.float32),
        mesh=_mesh(),
        scratch_types=[
            pltpu.VMEM((IB, 2, K), jnp.int32),
            pltpu.VMEM((K, d), jnp.float32),
            pltpu.VMEM((K, d), jnp.float32),
            pltpu.VMEM_SHARED((NPAD, d), jnp.float32),
            pltpu.SemaphoreType.DMA,
            pltpu.SemaphoreType.DMA,
        ],
        compiler_params=pltpu.CompilerParams(use_tc_tiling_on_sc=False),
    )
    def body(h_hbm, e_hbm, out_hbm, idx_v, rows0, rows1,
             acc, sem0, sem1):
        c = lax.axis_index("c")
        s = lax.axis_index("s")

        @pl.loop(0, K)
        def _(i):
            @pl.loop(0, d, step=16)
            def _(j):
                rows0[i, pl.ds(j, 16)] = jnp.zeros((16,), jnp.float32)

        @pl.loop(0, SLICE, step=K)
        def _(r):
            pltpu.sync_copy(rows0, acc.at[pl.ds(s * SLICE + r, K)])

        row0 = (c * NS + s) * ROWS_W
        plsc.subcore_barrier()

        @pl.loop(0, NCHUNK)
        def _(t):
            pltpu.sync_copy(e_hbm.at[pl.ds(row0 + t * IB, IB)], idx_v)
            pltpu.async_copy(h_hbm.at[idx_v.at[0, 0]], rows0, sem0)

            @pl.loop(0, IB - 2, step=2)
            def _(b):
                pltpu.async_copy(h_hbm.at[idx_v.at[b + 1, 0]], rows1, sem1)
                pltpu.make_async_copy(h_hbm.at[idx_v.at[b, 0]], rows0, sem0).wait()
                pltpu.sync_copy(rows0, acc.at[idx_v.at[b, 1]], add=True)
                pltpu.async_copy(h_hbm.at[idx_v.at[b + 2, 0]], rows0, sem0)
                pltpu.make_async_copy(h_hbm.at[idx_v.at[b + 1, 0]], rows1, sem1).wait()
                pltpu.sync_copy(rows1, acc.at[idx_v.at[b + 1, 1]], add=True)

            last = IB - 2
            pltpu.async_copy(h_hbm.at[idx_v.at[last + 1, 0]], rows1, sem1)
            pltpu.make_async_copy(h_hbm.at[idx_v.at[last, 0]], rows0, sem0).wait()
            pltpu.sync_copy(rows0, acc.at[idx_v.at[last, 1]], add=True)
            pltpu.make_async_copy(h_hbm.at[idx_v.at[last + 1, 0]], rows1, sem1).wait()
            pltpu.sync_copy(rows1, acc.at[idx_v.at[last + 1, 1]], add=True)

        plsc.subcore_barrier()
        pltpu.sync_copy(
            acc.at[pl.ds(s * SLICE, SLICE)],
            out_hbm.at[c].at[pl.ds(s * SLICE, SLICE)],
        )

    return body(hp, edges3d)


def _tc_mm(x, w):
    n, kdim = x.shape
    m = w.shape[1]

    def body(x_ref, w_ref, o_ref):
        o_ref[...] = jnp.dot(x_ref[...], w_ref[...],
                             preferred_element_type=jnp.float32)

    return pl.pallas_call(
        body,
        grid=(n // R,),
        in_specs=[
            pl.BlockSpec((R, kdim), lambda i: (i, 0)),
            pl.BlockSpec((kdim, m), lambda i: (0, 0)),
        ],
        out_specs=pl.BlockSpec((R, m), lambda i: (i, 0)),
        out_shape=jax.ShapeDtypeStruct((n, m), jnp.float32),
    )(x, w)


def _tc_scale(h, deg_parts):
    def body(h_ref, deg_ref, hp_ref, dinv_ref):
        deg = deg_ref[0, :, 0:1] + deg_ref[1, :, 0:1] + 1.0
        dinv = lax.rsqrt(deg)
        hp_ref[...] = h_ref[...] * dinv
        dinv_ref[...] = dinv

    return pl.pallas_call(
        body,
        grid=(N // R,),
        in_specs=[
            pl.BlockSpec((R, HID), lambda i: (i, 0)),
            pl.BlockSpec((2, R, DEG_W), lambda i: (0, i, 0)),
        ],
        out_specs=[
            pl.BlockSpec((R, HID), lambda i: (i, 0)),
            pl.BlockSpec((R, 1), lambda i: (i, 0)),
        ],
        out_shape=[
            jax.ShapeDtypeStruct((N, HID), jnp.float32),
            jax.ShapeDtypeStruct((N, 1), jnp.float32),
        ],
    )(h, deg_parts)


def _tc_layer2(s1, hp1, dinv, b1, w2):
    def body(s_ref, hp_ref, dinv_ref, b_ref, w_ref, o_ref):
        t = (s_ref[0] + s_ref[1] + hp_ref[...]) * dinv_ref[...] + b_ref[...]
        x2 = jnp.maximum(t, 0.0)
        h2 = jnp.dot(x2, w_ref[...], preferred_element_type=jnp.float32)
        o_ref[...] = h2 * dinv_ref[...]

    return pl.pallas_call(
        body,
        grid=(N // R,),
        in_specs=[
            pl.BlockSpec((2, R, HID), lambda i: (0, i, 0)),
            pl.BlockSpec((R, HID), lambda i: (i, 0)),
            pl.BlockSpec((R, 1), lambda i: (i, 0)),
            pl.BlockSpec((1, HID), lambda i: (0, 0)),
            pl.BlockSpec((HID, OUT), lambda i: (0, 0)),
        ],
        out_specs=pl.BlockSpec((R, OUT), lambda i: (i, 0)),
        out_shape=jax.ShapeDtypeStruct((N, OUT), jnp.float32),
    )(s1, hp1, dinv, b1, w2)


def _tc_out(s2, hp2, dinv, b2):
    def body(s_ref, hp_ref, dinv_ref, b_ref, o_ref):
        t = (s_ref[0] + s_ref[1] + hp_ref[...]) * dinv_ref[...] + b_ref[...]
        m = jnp.max(t, axis=1, keepdims=True)
        lse = jnp.log(jnp.sum(jnp.exp(t - m), axis=1, keepdims=True)) + m
        o_ref[...] = t - lse

    return pl.pallas_call(
        body,
        grid=(N // R,),
        in_specs=[
            pl.BlockSpec((2, R, OUT), lambda i: (0, i, 0)),
            pl.BlockSpec((R, OUT), lambda i: (i, 0)),
            pl.BlockSpec((R, 1), lambda i: (i, 0)),
            pl.BlockSpec((1, OUT), lambda i: (0, 0)),
        ],
        out_specs=pl.BlockSpec((R, OUT), lambda i: (i, 0)),
        out_shape=jax.ShapeDtypeStruct((N, OUT), jnp.float32),
    )(s2, hp2, dinv, b2)


@jax.jit
def kernel(x, edge_index, W1, b1, W2, b2):
    nw = NC * NS
    real_w = E // nw
    pad_w = EPAD // nw - real_w
    pad_src = jnp.zeros((nw, pad_w), jnp.int32)
    pad_dst = jnp.broadcast_to(
        N + (jnp.arange(pad_w, dtype=jnp.int32) % (NPAD - N)), (nw, pad_w))
    src2d = jnp.concatenate(
        [edge_index[0].reshape(nw, real_w), pad_src], axis=1).reshape(NB, K)
    dst2d = jnp.concatenate(
        [edge_index[1].reshape(nw, real_w), pad_dst], axis=1).reshape(NB, K)
    edges3d = jnp.stack([src2d, dst2d], axis=1)

    deg_parts = _sc_deg(edges3d)
    h1 = _tc_mm(x, W1)
    hp1, dinv = _tc_scale(h1, deg_parts)

    s1 = _sc_gs(hp1, edges3d, HID)
    hp2 = _tc_layer2(s1, hp1, dinv, b1.reshape(1, HID), W2)

    s2 = _sc_gs(hp2, edges3d, OUT)
    return _tc_out(s2, hp2, dinv, b2.reshape(1, OUT))


# speedup vs baseline: 1.0034x; 1.0034x over previous
import functools

import jax
import jax.numpy as jnp
from jax import lax
from jax.experimental import pallas as pl
from jax.experimental.pallas import tpu as pltpu
from jax.experimental.pallas import tpu_sc as plsc

N = 10000
E = 320000
IN_DIM = 128
HID = 128
OUT = 64

NPAD = 10240
K = 128
NC = 2
NS = 16
ROWS_W = 80
NB = NC * NS * ROWS_W
EPAD = NB * K
SLICE = NPAD // NS
DEG_W = 16
IB = 40
NCHUNK = ROWS_W // IB
NBUF = 2
R = 1000


def _mesh():
    return plsc.VectorSubcoreMesh(core_axis_name="c", subcore_axis_name="s")


def _sc_deg(edges3d):
    @functools.partial(
        pl.kernel,
        out_type=jax.ShapeDtypeStruct((NC, NPAD, DEG_W), jnp.float32),
        mesh=_mesh(),
        scratch_types=[
            pltpu.VMEM((ROWS_W, 2, K), jnp.int32),
            pltpu.VMEM((K, DEG_W), jnp.float32),
            pltpu.VMEM_SHARED((NPAD, DEG_W), jnp.float32),
        ],
    )
    def body(e_hbm, out_hbm, idx_v, buf_v, acc):
        c = lax.axis_index("c")
        s = lax.axis_index("s")

        @pl.loop(0, K)
        def _(i):
            buf_v[i, :] = jnp.zeros((DEG_W,), jnp.float32)

        @pl.loop(0, SLICE, step=K)
        def _(r):
            pltpu.sync_copy(buf_v, acc.at[pl.ds(s * SLICE + r, K)])

        row0 = (c * NS + s) * ROWS_W
        pltpu.sync_copy(e_hbm.at[pl.ds(row0, ROWS_W)], idx_v)

        @pl.loop(0, K)
        def _(i):
            buf_v[i, :] = jnp.ones((DEG_W,), jnp.float32)

        plsc.subcore_barrier()

        @pl.loop(0, ROWS_W)
        def _(b):
            pltpu.sync_copy(buf_v, acc.at[idx_v.at[b, 1]], add=True)

        plsc.subcore_barrier()
        pltpu.sync_copy(
            acc.at[pl.ds(s * SLICE, SLICE)],
            out_hbm.at[c].at[pl.ds(s * SLICE, SLICE)],
        )

    return body(edges3d)


def _sc_gs(hp, edges3d, d):
    @functools.partial(
        pl.kernel,
        out_type=jax.ShapeDtypeStruct((NC, NPAD, d), jnp.float32),
        mesh=_mesh(),
        scratch_types=[
            pltpu.VMEM((IB, 2, K), jnp.int32),
            pltpu.VMEM((K, d), jnp.float32),
            pltpu.VMEM((K, d), jnp.float32),
            pltpu.VMEM_SHARED((NPAD, d), jnp.float32),
            pltpu.SemaphoreType.DMA,
            pltpu.SemaphoreType.DMA,
        ],
        compiler_params=pltpu.CompilerParams(use_tc_tiling_on_sc=False),
    )
    def body(h_hbm, e_hbm, out_hbm, idx_v, rows0, rows1,
             acc, sem0, sem1):
        c = lax.axis_index("c")
        s = lax.axis_index("s")

        @pl.loop(0, K)
        def _(i):
            @pl.loop(0, d, step=16)
            def _(j):
                rows0[i, pl.ds(j, 16)] = jnp.zeros((16,), jnp.float32)

        @pl.loop(0, SLICE, step=K)
        def _(r):
            pltpu.sync_copy(rows0, acc.at[pl.ds(s * SLICE + r, K)])

        row0 = (c * NS + s) * ROWS_W
        plsc.subcore_barrier()

        @pl.loop(0, NCHUNK)
        def _(t):
            pltpu.sync_copy(e_hbm.at[pl.ds(row0 + t * IB, IB)], idx_v)
            pltpu.async_copy(h_hbm.at[idx_v.at[0, 0]], rows0, sem0)

            @pl.loop(0, IB - 2, step=2)
            def _(b):
                pltpu.async_copy(h_hbm.at[idx_v.at[b + 1, 0]], rows1, sem1)
                pltpu.make_async_copy(h_hbm.at[idx_v.at[b, 0]], rows0, sem0).wait()
                pltpu.sync_copy(rows0, acc.at[idx_v.at[b, 1]], add=True)
                pltpu.async_copy(h_hbm.at[idx_v.at[b + 2, 0]], rows0, sem0)
                pltpu.make_async_copy(h_hbm.at[idx_v.at[b + 1, 0]], rows1, sem1).wait()
                pltpu.sync_copy(rows1, acc.at[idx_v.at[b + 1, 1]], add=True)

            last = IB - 2
            pltpu.async_copy(h_hbm.at[idx_v.at[last + 1, 0]], rows1, sem1)
            pltpu.make_async_copy(h_hbm.at[idx_v.at[last, 0]], rows0, sem0).wait()
            pltpu.sync_copy(rows0, acc.at[idx_v.at[last, 1]], add=True)
            pltpu.make_async_copy(h_hbm.at[idx_v.at[last + 1, 0]], rows1, sem1).wait()
            pltpu.sync_copy(rows1, acc.at[idx_v.at[last + 1, 1]], add=True)

        plsc.subcore_barrier()
        pltpu.sync_copy(
            acc.at[pl.ds(s * SLICE, SLICE)],
            out_hbm.at[c].at[pl.ds(s * SLICE, SLICE)],
        )

    return body(hp, edges3d)


def _tc_mm(x, w):
    n, kdim = x.shape
    m = w.shape[1]

    def body(x_ref, w_ref, o_ref):
        o_ref[...] = jnp.dot(x_ref[...], w_ref[...],
                             preferred_element_type=jnp.float32)

    return pl.pallas_call(
        body,
        grid=(n // R,),
        in_specs=[
            pl.BlockSpec((R, kdim), lambda i: (i, 0)),
            pl.BlockSpec((kdim, m), lambda i: (0, 0)),
        ],
        out_specs=pl.BlockSpec((R, m), lambda i: (i, 0)),
        out_shape=jax.ShapeDtypeStruct((n, m), jnp.float32),
    )(x, w)


def _tc_scale(h, deg_parts):
    def body(h_ref, deg_ref, hp_ref, dinv_ref):
        deg = deg_ref[0, :, 0:1] + deg_ref[1, :, 0:1] + 1.0
        dinv = lax.rsqrt(deg)
        hp_ref[...] = h_ref[...] * dinv
        dinv_ref[...] = dinv

    return pl.pallas_call(
        body,
        grid=(N // R,),
        in_specs=[
            pl.BlockSpec((R, HID), lambda i: (i, 0)),
            pl.BlockSpec((2, R, DEG_W), lambda i: (0, i, 0)),
        ],
        out_specs=[
            pl.BlockSpec((R, HID), lambda i: (i, 0)),
            pl.BlockSpec((R, 1), lambda i: (i, 0)),
        ],
        out_shape=[
            jax.ShapeDtypeStruct((N, HID), jnp.float32),
            jax.ShapeDtypeStruct((N, 1), jnp.float32),
        ],
    )(h, deg_parts)


def _tc_layer2(s1, hp1, dinv, b1, w2):
    def body(s_ref, hp_ref, dinv_ref, b_ref, w_ref, o_ref):
        t = (s_ref[0] + s_ref[1] + hp_ref[...]) * dinv_ref[...] + b_ref[...]
        x2 = jnp.maximum(t, 0.0)
        h2 = jnp.dot(x2, w_ref[...], preferred_element_type=jnp.float32)
        o_ref[...] = h2 * dinv_ref[...]

    return pl.pallas_call(
        body,
        grid=(N // R,),
        in_specs=[
            pl.BlockSpec((2, R, HID), lambda i: (0, i, 0)),
            pl.BlockSpec((R, HID), lambda i: (i, 0)),
            pl.BlockSpec((R, 1), lambda i: (i, 0)),
            pl.BlockSpec((1, HID), lambda i: (0, 0)),
            pl.BlockSpec((HID, OUT), lambda i: (0, 0)),
        ],
        out_specs=pl.BlockSpec((R, OUT), lambda i: (i, 0)),
        out_shape=jax.ShapeDtypeStruct((N, OUT), jnp.float32),
    )(s1, hp1, dinv, b1, w2)


def _tc_out(s2, hp2, dinv, b2):
    def body(s_ref, hp_ref, dinv_ref, b_ref, o_ref):
        t = (s_ref[0] + s_ref[1] + hp_ref[...]) * dinv_ref[...] + b_ref[...]
        m = jnp.max(t, axis=1, keepdims=True)
        lse = jnp.log(jnp.sum(jnp.exp(t - m), axis=1, keepdims=True)) + m
        o_ref[...] = t - lse

    return pl.pallas_call(
        body,
        grid=(N // R,),
        in_specs=[
            pl.BlockSpec((2, R, OUT), lambda i: (0, i, 0)),
            pl.BlockSpec((R, OUT), lambda i: (i, 0)),
            pl.BlockSpec((R, 1), lambda i: (i, 0)),
            pl.BlockSpec((1, OUT), lambda i: (0, 0)),
        ],
        out_specs=pl.BlockSpec((R, OUT), lambda i: (i, 0)),
        out_shape=jax.ShapeDtypeStruct((N, OUT), jnp.float32),
    )(s2, hp2, dinv, b2)


@jax.jit
def kernel(x, edge_index, W1, b1, W2, b2):
    npad_e = EPAD - E
    pad_src = jnp.zeros((npad_e,), jnp.int32)
    pad_dst = N + (jnp.arange(npad_e, dtype=jnp.int32) % (NPAD - N))
    src2d = jnp.concatenate([edge_index[0], pad_src]).reshape(NB, K)
    dst2d = jnp.concatenate([edge_index[1], pad_dst]).reshape(NB, K)
    edges3d = jnp.stack([src2d, dst2d], axis=1)

    deg_parts = _sc_deg(edges3d)
    h1 = _tc_mm(x, W1)
    hp1, dinv = _tc_scale(h1, deg_parts)

    s1 = _sc_gs(hp1, edges3d, HID)
    hp2 = _tc_layer2(s1, hp1, dinv, b1.reshape(1, HID), W2)

    s2 = _sc_gs(hp2, edges3d, OUT)
    return _tc_out(s2, hp2, dinv, b2.reshape(1, OUT))
